# baseline (speedup 1.0000x reference)
import functools

import jax
import jax.numpy as jnp
from jax import lax
from jax.experimental import pallas as pl
from jax.experimental.pallas import tpu as pltpu
from jax.experimental.pallas import tpu_sc as plsc

B = 1024
C_IN = 26
C_OUT = 36
N, M = 32, 32
NC, NS, L = 2, 16, 16
NW = NC * NS
B_PER_W = B // NW
NVEC = (N * M) // L
NBUF = 3
GEN_OFF = 28.0
OCC_OFF = 32.0
CTAB = 200


def _fire_out(in_buf, emb_buf, out_hbm, k, m, sem):
    pltpu.make_async_copy(in_buf.at[k, pl.ds(0, 18)],
                          out_hbm.at[m, pl.ds(0, 18)], sem).start()
    pltpu.make_async_copy(in_buf.at[k, pl.ds(19, 2)],
                          out_hbm.at[m, pl.ds(18, 2)], sem).start()
    pltpu.make_async_copy(in_buf.at[k, pl.ds(24, 2)],
                          out_hbm.at[m, pl.ds(34, 2)], sem).start()
    pltpu.make_async_copy(emb_buf.at[k], out_hbm.at[m, pl.ds(20, 14)], sem).start()


def _drain_out(in_buf, emb_buf, out_hbm, k, m, sem):
    pltpu.make_async_copy(in_buf.at[k, pl.ds(0, 18)],
                          out_hbm.at[m, pl.ds(0, 18)], sem).wait()
    pltpu.make_async_copy(in_buf.at[k, pl.ds(19, 2)],
                          out_hbm.at[m, pl.ds(18, 2)], sem).wait()
    pltpu.make_async_copy(in_buf.at[k, pl.ds(24, 2)],
                          out_hbm.at[m, pl.ds(34, 2)], sem).wait()
    pltpu.make_async_copy(emb_buf.at[k], out_hbm.at[m, pl.ds(20, 14)], sem).wait()


def _sc_body(x_hbm, ctab_hbm, out_hbm, ctab_v, in_buf, emb_buf,
             si0, si1, si2, so0, so1, so2):
    si = (si0, si1, si2)
    so = (so0, so1, so2)
    c = lax.axis_index("c")
    s = lax.axis_index("s")
    wid = s * NC + c
    base = wid * B_PER_W

    pltpu.sync_copy(ctab_hbm, ctab_v)

    pltpu.async_copy(x_hbm.at[base], in_buf.at[0], si[0])

    def compute(k):
        def per_vec(v, carry):
            r = v // 2
            sl = pl.ds((v % 2) * L, L)
            av = in_buf[k, 21, r, sl]
            gv = in_buf[k, 22, r, sl]
            ov = in_buf[k, 23, r, sl]
            ab = (av * 4.0).astype(jnp.int32)
            gb = (gv * 2.0 + GEN_OFF).astype(jnp.int32)
            ob = (ov * 8.0 + OCC_OFF).astype(jnp.int32)
            emb_buf[k, 0, r, sl] = plsc.load_gather(ctab_v, [ab])
            for d in range(1, 4):
                emb_buf[k, d, r, sl] = plsc.load_gather(ctab_v, [ab + d])
            emb_buf[k, 4, r, sl] = plsc.load_gather(ctab_v, [gb])
            emb_buf[k, 5, r, sl] = plsc.load_gather(ctab_v, [gb + 1])
            emb_buf[k, 6, r, sl] = plsc.load_gather(ctab_v, [ob])
            for d in range(1, 8):
                emb_buf[k, 6 + d, r, sl] = plsc.load_gather(ctab_v, [ob + d])
            return carry

        lax.fori_loop(0, NVEC, per_vec, 0, unroll=4)

    def per_turn(g, carry):
        for k in range(NBUF):
            i = g * NBUF + k
            m = base + i

            k2 = (k + 1) % NBUF

            @pl.when(i >= 2)
            def _():
                _drain_out(in_buf, emb_buf, out_hbm, k2, m - 2, so[k2])

            @pl.when(i + 1 < B_PER_W)
            def _():
                pltpu.async_copy(x_hbm.at[m + 1], in_buf.at[k2], si[k2])

            @pl.when(i < B_PER_W)
            def _():
                pltpu.make_async_copy(x_hbm.at[m], in_buf.at[k], si[k]).wait()
                compute(k)
                _fire_out(in_buf, emb_buf, out_hbm, k, m, so[k])

        return carry

    lax.fori_loop(0, (B_PER_W + NBUF) // NBUF, per_turn, 0)

    _drain_out(in_buf, emb_buf, out_hbm, 1, base + B_PER_W - 1, so[1])


@jax.jit
def kernel(x, age_table, gender_table, occupation_table):
    ctab = jnp.concatenate([age_table.reshape(-1), gender_table.reshape(-1),
                            occupation_table.reshape(-1)])
    mesh = plsc.VectorSubcoreMesh(core_axis_name="c", subcore_axis_name="s",
                                  num_cores=NC, num_subcores=NS)
    out = pl.kernel(
        _sc_body,
        out_type=jax.ShapeDtypeStruct((B, C_OUT, N, M), jnp.float32),
        mesh=mesh,
        scratch_types=[
            pltpu.VMEM((CTAB,), jnp.float32),
            pltpu.VMEM((NBUF, C_IN, N, M), jnp.float32),
            pltpu.VMEM((NBUF, 14, N, M), jnp.float32),
            pltpu.SemaphoreType.DMA,
            pltpu.SemaphoreType.DMA,
            pltpu.SemaphoreType.DMA,
            pltpu.SemaphoreType.DMA,
            pltpu.SemaphoreType.DMA,
            pltpu.SemaphoreType.DMA,
        ],
        compiler_params=pltpu.CompilerParams(use_tc_tiling_on_sc=False,
                                             needs_layout_passes=False),
    )(x, ctab)
    return out


# speedup vs baseline: 3.7089x; 3.7089x over previous
import functools

import jax
import jax.numpy as jnp
from jax import lax
from jax.experimental import pallas as pl
from jax.experimental.pallas import tpu as pltpu
from jax.experimental.pallas import tpu_sc as plsc

B = 1024
C_IN = 26
C_OUT = 36
NM = 1024
NC, NS, L = 2, 16, 16
NW = NC * NS
R_PER_W = NM // NW
RSUB = 8
BSUB = 512
NITER = (R_PER_W // RSUB) * (B // BSUB)
NVEC = (RSUB * BSUB) // L
GEN_OFF = 28.0
OCC_OFF = 32.0
CTAB = 200

PASS_GROUPS = ((0, 0, 9), (9, 9, 9), (19, 18, 2), (24, 34, 2))


def _sc_body(x_hbm, ctab_hbm, out_hbm, ctab_v, idx_v, emb_v, pass_v):
    c = lax.axis_index("c")
    s = lax.axis_index("s")
    wid = s * NC + c
    base_r = wid * R_PER_W

    pltpu.sync_copy(ctab_hbm, ctab_v)

    def per_block(t, carry):
        r0 = base_r + (t // 2) * RSUB
        b0 = (t % 2) * BSUB
        rs = pl.ds(r0, RSUB)
        bs = pl.ds(b0, BSUB)

        pltpu.sync_copy(x_hbm.at[pl.ds(21, 3), rs, bs], idx_v)

        def per_vec(v, carry2):
            r = v // (BSUB // L)
            sl = pl.ds((v % (BSUB // L)) * L, L)
            av = idx_v[0, r, sl]
            gv = idx_v[1, r, sl]
            ov = idx_v[2, r, sl]
            ab = (av * 4.0).astype(jnp.int32)
            gb = (gv * 2.0 + GEN_OFF).astype(jnp.int32)
            ob = (ov * 8.0 + OCC_OFF).astype(jnp.int32)
            emb_v[0, r, sl] = plsc.load_gather(ctab_v, [ab])
            for d in range(1, 4):
                emb_v[d, r, sl] = plsc.load_gather(ctab_v, [ab + d])
            emb_v[4, r, sl] = plsc.load_gather(ctab_v, [gb])
            emb_v[5, r, sl] = plsc.load_gather(ctab_v, [gb + 1])
            emb_v[6, r, sl] = plsc.load_gather(ctab_v, [ob])
            for d in range(1, 8):
                emb_v[6 + d, r, sl] = plsc.load_gather(ctab_v, [ob + d])
            return carry2

        lax.fori_loop(0, NVEC, per_vec, 0, unroll=4)

        pltpu.sync_copy(emb_v, out_hbm.at[pl.ds(20, 14), rs, bs])

        for sc0, dc0, n in PASS_GROUPS:
            pv = pass_v.at[pl.ds(0, n)]
            pltpu.sync_copy(x_hbm.at[pl.ds(sc0, n), rs, bs], pv)
            pltpu.sync_copy(pv, out_hbm.at[pl.ds(dc0, n), rs, bs])
        return carry

    lax.fori_loop(0, NITER, per_block, 0)


@jax.jit
def kernel(x, age_table, gender_table, occupation_table):
    x_t = jnp.transpose(x.reshape(B, C_IN, NM), (1, 2, 0))
    ctab = jnp.concatenate([age_table.reshape(-1), gender_table.reshape(-1),
                            occupation_table.reshape(-1)])
    mesh = plsc.VectorSubcoreMesh(core_axis_name="c", subcore_axis_name="s",
                                  num_cores=NC, num_subcores=NS)
    out_t = pl.kernel(
        _sc_body,
        out_type=jax.ShapeDtypeStruct((C_OUT, NM, B), jnp.float32),
        mesh=mesh,
        scratch_types=[
            pltpu.VMEM((CTAB,), jnp.float32),
            pltpu.VMEM((3, RSUB, BSUB), jnp.float32),
            pltpu.VMEM((14, RSUB, BSUB), jnp.float32),
            pltpu.VMEM((9, RSUB, BSUB), jnp.float32),
        ],
        compiler_params=pltpu.CompilerParams(use_tc_tiling_on_sc=True,
                                             needs_layout_passes=False),
    )(x_t, ctab)
    return jnp.transpose(out_t, (2, 0, 1)).reshape(B, C_OUT, 32, 32)


# speedup vs baseline: 5.6941x; 1.5352x over previous
import functools

import jax
import jax.numpy as jnp
from jax import lax
from jax.experimental import pallas as pl
from jax.experimental.pallas import tpu as pltpu
from jax.experimental.pallas import tpu_sc as plsc

B = 1024
C_IN = 26
C_OUT = 36
NM = 1024
NC, NS, L = 2, 16, 16
NW = NC * NS
R_PER_W = NM // NW
RSUB = 8
BSUB = 128
NBLK_R = R_PER_W // RSUB
NBLK_B = B // BSUB
NITER = NBLK_R * NBLK_B
NVEC = (RSUB * BSUB) // L
VPR = BSUB // L
NBUF = 3
GEN_OFF = 28.0
OCC_OFF = 32.0
CTAB = 200

PASS_IN = ((0, 0, 18, 0), (19, 18, 2, 18), (24, 34, 2, 20))


def _fire_in(x_hbm, idx_v, pass_v, k, rs, bs, sem):
    pltpu.make_async_copy(x_hbm.at[pl.ds(21, 3), rs, bs], idx_v.at[k], sem).start()
    for sc0, _, n, po in PASS_IN:
        pltpu.make_async_copy(x_hbm.at[pl.ds(sc0, n), rs, bs],
                              pass_v.at[k, pl.ds(po, n)], sem).start()


def _wait_in(x_hbm, idx_v, pass_v, k, rs, bs, sem):
    pltpu.make_async_copy(x_hbm.at[pl.ds(21, 3), rs, bs], idx_v.at[k], sem).wait()
    for sc0, _, n, po in PASS_IN:
        pltpu.make_async_copy(x_hbm.at[pl.ds(sc0, n), rs, bs],
                              pass_v.at[k, pl.ds(po, n)], sem).wait()


def _fire_out(out_hbm, emb_v, pass_v, k, rs, bs, sem):
    pltpu.make_async_copy(emb_v.at[k], out_hbm.at[pl.ds(20, 14), rs, bs], sem).start()
    for _, dc0, n, po in PASS_IN:
        pltpu.make_async_copy(pass_v.at[k, pl.ds(po, n)],
                              out_hbm.at[pl.ds(dc0, n), rs, bs], sem).start()


def _drain_out(out_hbm, emb_v, pass_v, k, rs, bs, sem):
    pltpu.make_async_copy(emb_v.at[k], out_hbm.at[pl.ds(20, 14), rs, bs], sem).wait()
    for _, dc0, n, po in PASS_IN:
        pltpu.make_async_copy(pass_v.at[k, pl.ds(po, n)],
                              out_hbm.at[pl.ds(dc0, n), rs, bs], sem).wait()


def _block_slices(base_r, t):
    rs = pl.ds(base_r + (t // NBLK_B) * RSUB, RSUB)
    bs = pl.ds((t % NBLK_B) * BSUB, BSUB)
    return rs, bs


def _sc_body(x_hbm, ctab_hbm, out_hbm, ctab_v, idx_v, emb_v, pass_v,
             si0, si1, si2, so0, so1, so2):
    si = (si0, si1, si2)
    so = (so0, so1, so2)
    c = lax.axis_index("c")
    s = lax.axis_index("s")
    wid = s * NC + c
    base_r = wid * R_PER_W

    pltpu.sync_copy(ctab_hbm, ctab_v)

    rs0, bs0 = _block_slices(base_r, 0)
    _fire_in(x_hbm, idx_v, pass_v, 0, rs0, bs0, si[0])

    def compute(k):
        def per_vec(v, carry):
            r = v // VPR
            sl = pl.ds((v % VPR) * L, L)
            av = idx_v[k, 0, r, sl]
            gv = idx_v[k, 1, r, sl]
            ov = idx_v[k, 2, r, sl]
            ab = (av * 4.0).astype(jnp.int32)
            gb = (gv * 2.0 + GEN_OFF).astype(jnp.int32)
            ob = (ov * 8.0 + OCC_OFF).astype(jnp.int32)
            emb_v[k, 0, r, sl] = plsc.load_gather(ctab_v, [ab])
            for d in range(1, 4):
                emb_v[k, d, r, sl] = plsc.load_gather(ctab_v, [ab + d])
            emb_v[k, 4, r, sl] = plsc.load_gather(ctab_v, [gb])
            emb_v[k, 5, r, sl] = plsc.load_gather(ctab_v, [gb + 1])
            emb_v[k, 6, r, sl] = plsc.load_gather(ctab_v, [ob])
            for d in range(1, 8):
                emb_v[k, 6 + d, r, sl] = plsc.load_gather(ctab_v, [ob + d])
            return carry

        lax.fori_loop(0, NVEC, per_vec, 0, unroll=4)

    def per_turn(g, carry):
        for k in range(NBUF):
            t = g * NBUF + k

            k2 = (k + 1) % NBUF

            @pl.when(jnp.logical_and(t >= 2, t - 2 < NITER))
            def _():
                rs, bs = _block_slices(base_r, t - 2)
                _drain_out(out_hbm, emb_v, pass_v, k2, rs, bs, so[k2])

            @pl.when(t + 1 < NITER)
            def _():
                rs, bs = _block_slices(base_r, t + 1)
                _fire_in(x_hbm, idx_v, pass_v, k2, rs, bs, si[k2])

            @pl.when(t < NITER)
            def _():
                rs, bs = _block_slices(base_r, t)
                _wait_in(x_hbm, idx_v, pass_v, k, rs, bs, si[k])
                compute(k)
                _fire_out(out_hbm, emb_v, pass_v, k, rs, bs, so[k])

        return carry

    lax.fori_loop(0, (NITER + NBUF) // NBUF, per_turn, 0)


@jax.jit
def kernel(x, age_table, gender_table, occupation_table):
    x_t = jnp.transpose(x.reshape(B, C_IN, NM), (1, 2, 0))
    ctab = jnp.concatenate([age_table.reshape(-1), gender_table.reshape(-1),
                            occupation_table.reshape(-1)])
    mesh = plsc.VectorSubcoreMesh(core_axis_name="c", subcore_axis_name="s",
                                  num_cores=NC, num_subcores=NS)
    out_t = pl.kernel(
        _sc_body,
        out_type=jax.ShapeDtypeStruct((C_OUT, NM, B), jnp.float32),
        mesh=mesh,
        scratch_types=[
            pltpu.VMEM((CTAB,), jnp.float32),
            pltpu.VMEM((NBUF, 3, RSUB, BSUB), jnp.float32),
            pltpu.VMEM((NBUF, 14, RSUB, BSUB), jnp.float32),
            pltpu.VMEM((NBUF, 22, RSUB, BSUB), jnp.float32),
            pltpu.SemaphoreType.DMA,
            pltpu.SemaphoreType.DMA,
            pltpu.SemaphoreType.DMA,
            pltpu.SemaphoreType.DMA,
            pltpu.SemaphoreType.DMA,
            pltpu.SemaphoreType.DMA,
        ],
        compiler_params=pltpu.CompilerParams(use_tc_tiling_on_sc=True,
                                             needs_layout_passes=False),
    )(x_t, ctab)
    return jnp.transpose(out_t, (2, 0, 1)).reshape(B, C_OUT, 32, 32)


# speedup vs baseline: 5.9410x; 1.0434x over previous
import functools

import jax
import jax.numpy as jnp
from jax import lax
from jax.experimental import pallas as pl
from jax.experimental.pallas import tpu as pltpu
from jax.experimental.pallas import tpu_sc as plsc

B = 1024
C_IN = 26
C_OUT = 36
NM = 1024
NC, NS, L = 2, 16, 16
NW = NC * NS
R_PER_W = NM // NW
RSUB = 8
BSUB = 128
NBLK_R = R_PER_W // RSUB
NBLK_B = B // BSUB
NITER = NBLK_R * NBLK_B
NVEC = (RSUB * BSUB) // L
VPR = BSUB // L
NBUF = 3
GEN_OFF = 28.0
OCC_OFF = 32.0
CTAB = 200

PASS_OUT = ((0, 0, 18), (19, 18, 2), (24, 34, 2))


def _fire_in(x_hbm, in_v, k, rs, bs, sem):
    pltpu.make_async_copy(x_hbm.at[:, rs, bs], in_v.at[k], sem).start()


def _wait_in(x_hbm, in_v, k, rs, bs, sem):
    pltpu.make_async_copy(x_hbm.at[:, rs, bs], in_v.at[k], sem).wait()


def _fire_out(out_hbm, emb_v, in_v, k, rs, bs, sem):
    pltpu.make_async_copy(emb_v.at[k], out_hbm.at[pl.ds(20, 14), rs, bs], sem).start()
    for sc0, dc0, n in PASS_OUT:
        pltpu.make_async_copy(in_v.at[k, pl.ds(sc0, n)],
                              out_hbm.at[pl.ds(dc0, n), rs, bs], sem).start()


def _drain_out(out_hbm, emb_v, in_v, k, rs, bs, sem):
    pltpu.make_async_copy(emb_v.at[k], out_hbm.at[pl.ds(20, 14), rs, bs], sem).wait()
    for sc0, dc0, n in PASS_OUT:
        pltpu.make_async_copy(in_v.at[k, pl.ds(sc0, n)],
                              out_hbm.at[pl.ds(dc0, n), rs, bs], sem).wait()


def _block_slices(base_r, t):
    rs = pl.ds(base_r + (t // NBLK_B) * RSUB, RSUB)
    bs = pl.ds((t % NBLK_B) * BSUB, BSUB)
    return rs, bs


def _sc_body(x_hbm, ctab_hbm, out_hbm, ctab_v, in_v, emb_v,
             si0, si1, si2, so0, so1, so2):
    si = (si0, si1, si2)
    so = (so0, so1, so2)
    c = lax.axis_index("c")
    s = lax.axis_index("s")
    wid = s * NC + c
    base_r = wid * R_PER_W

    pltpu.sync_copy(ctab_hbm, ctab_v)

    rs0, bs0 = _block_slices(base_r, 0)
    _fire_in(x_hbm, in_v, 0, rs0, bs0, si[0])

    def compute(k):
        def per_vec(v, carry):
            r = v // VPR
            sl = pl.ds((v % VPR) * L, L)
            av = in_v[k, 21, r, sl]
            gv = in_v[k, 22, r, sl]
            ov = in_v[k, 23, r, sl]
            ab = (av * 4.0).astype(jnp.int32)
            gb = (gv * 2.0 + GEN_OFF).astype(jnp.int32)
            ob = (ov * 8.0 + OCC_OFF).astype(jnp.int32)
            emb_v[k, 0, r, sl] = plsc.load_gather(ctab_v, [ab])
            for d in range(1, 4):
                emb_v[k, d, r, sl] = plsc.load_gather(ctab_v, [ab + d])
            emb_v[k, 4, r, sl] = plsc.load_gather(ctab_v, [gb])
            emb_v[k, 5, r, sl] = plsc.load_gather(ctab_v, [gb + 1])
            emb_v[k, 6, r, sl] = plsc.load_gather(ctab_v, [ob])
            for d in range(1, 8):
                emb_v[k, 6 + d, r, sl] = plsc.load_gather(ctab_v, [ob + d])
            return carry

        lax.fori_loop(0, NVEC, per_vec, 0, unroll=8)

    def per_turn(g, carry):
        for k in range(NBUF):
            t = g * NBUF + k

            k2 = (k + 1) % NBUF

            @pl.when(jnp.logical_and(t >= 2, t - 2 < NITER))
            def _():
                rs, bs = _block_slices(base_r, t - 2)
                _drain_out(out_hbm, emb_v, in_v, k2, rs, bs, so[k2])

            @pl.when(t + 1 < NITER)
            def _():
                rs, bs = _block_slices(base_r, t + 1)
                _fire_in(x_hbm, in_v, k2, rs, bs, si[k2])

            @pl.when(t < NITER)
            def _():
                rs, bs = _block_slices(base_r, t)
                _wait_in(x_hbm, in_v, k, rs, bs, si[k])
                compute(k)
                _fire_out(out_hbm, emb_v, in_v, k, rs, bs, so[k])

        return carry

    lax.fori_loop(0, (NITER + NBUF) // NBUF, per_turn, 0)


@jax.jit
def kernel(x, age_table, gender_table, occupation_table):
    x_t = jnp.transpose(x.reshape(B, C_IN, NM), (1, 2, 0))
    ctab = jnp.concatenate([age_table.reshape(-1), gender_table.reshape(-1),
                            occupation_table.reshape(-1)])
    mesh = plsc.VectorSubcoreMesh(core_axis_name="c", subcore_axis_name="s",
                                  num_cores=NC, num_subcores=NS)
    out_t = pl.kernel(
        _sc_body,
        out_type=jax.ShapeDtypeStruct((C_OUT, NM, B), jnp.float32),
        mesh=mesh,
        scratch_types=[
            pltpu.VMEM((CTAB,), jnp.float32),
            pltpu.VMEM((NBUF, C_IN, RSUB, BSUB), jnp.float32),
            pltpu.VMEM((NBUF, 14, RSUB, BSUB), jnp.float32),
            pltpu.SemaphoreType.DMA,
            pltpu.SemaphoreType.DMA,
            pltpu.SemaphoreType.DMA,
            pltpu.SemaphoreType.DMA,
            pltpu.SemaphoreType.DMA,
            pltpu.SemaphoreType.DMA,
        ],
        compiler_params=pltpu.CompilerParams(use_tc_tiling_on_sc=True,
                                             needs_layout_passes=False),
    )(x_t, ctab)
    return jnp.transpose(out_t, (2, 0, 1)).reshape(B, C_OUT, 32, 32)


# speedup vs baseline: 9.6577x; 1.6256x over previous
import functools

import jax
import jax.numpy as jnp
from jax import lax
from jax.experimental import pallas as pl
from jax.experimental.pallas import tpu as pltpu
from jax.experimental.pallas import tpu_sc as plsc

B = 1024
C_IN = 26
C_OUT = 36
NM = 1024
NC, NS, L = 2, 16, 16
NW = NC * NS
R_PER_W = NM // NW
RSUB = 8
BSUB = 128
NBLK_R = R_PER_W // RSUB
NBLK_B = B // BSUB
NITER = NBLK_R * NBLK_B
NVEC = (RSUB * BSUB) // L
VPR = BSUB // L
NBUF = 3
GEN_OFF = 28.0
OCC_OFF = 32.0
CTAB = 200

PASS_OUT = ((0, 0, 18), (19, 18, 2), (24, 34, 2))


def _fire_in(x_hbm, in_v, k, rs, bs, sem):
    pltpu.make_async_copy(x_hbm.at[:, rs, bs], in_v.at[k], sem).start()


def _wait_in(x_hbm, in_v, k, rs, bs, sem):
    pltpu.make_async_copy(x_hbm.at[:, rs, bs], in_v.at[k], sem).wait()


def _fire_out(out_hbm, emb_v, in_v, k, rs, bs, sem):
    pltpu.make_async_copy(emb_v.at[k], out_hbm.at[pl.ds(20, 14), rs, bs], sem).start()
    for sc0, dc0, n in PASS_OUT:
        pltpu.make_async_copy(in_v.at[k, pl.ds(sc0, n)],
                              out_hbm.at[pl.ds(dc0, n), rs, bs], sem).start()


def _drain_out(out_hbm, emb_v, in_v, k, rs, bs, sem):
    pltpu.make_async_copy(emb_v.at[k], out_hbm.at[pl.ds(20, 14), rs, bs], sem).wait()
    for sc0, dc0, n in PASS_OUT:
        pltpu.make_async_copy(in_v.at[k, pl.ds(sc0, n)],
                              out_hbm.at[pl.ds(dc0, n), rs, bs], sem).wait()


def _block_slices(base_r, t):
    rs = pl.ds(base_r + (t // NBLK_B) * RSUB, RSUB)
    bs = pl.ds((t % NBLK_B) * BSUB, BSUB)
    return rs, bs


def _sc_body(x_hbm, ctab_hbm, out_hbm, ctab_v, in_v, emb_v,
             si0, si1, si2, so0, so1, so2):
    si = (si0, si1, si2)
    so = (so0, so1, so2)
    c = lax.axis_index("c")
    s = lax.axis_index("s")
    wid = s * NC + c
    base_r = wid * R_PER_W

    pltpu.sync_copy(ctab_hbm, ctab_v)

    rs0, bs0 = _block_slices(base_r, 0)
    _fire_in(x_hbm, in_v, 0, rs0, bs0, si[0])

    def compute(k):
        @plsc.parallel_loop(0, NVEC, 1, unroll=8)
        def per_vec(v):
            r = v // VPR
            sl = pl.ds((v % VPR) * L, L)
            av = in_v[k, 21, r, sl]
            gv = in_v[k, 22, r, sl]
            ov = in_v[k, 23, r, sl]
            ab = (av * 4.0).astype(jnp.int32)
            gb = (gv * 2.0 + GEN_OFF).astype(jnp.int32)
            ob = (ov * 8.0 + OCC_OFF).astype(jnp.int32)
            emb_v[k, 0, r, sl] = plsc.load_gather(ctab_v, [ab])
            for d in range(1, 4):
                emb_v[k, d, r, sl] = plsc.load_gather(ctab_v, [ab + d])
            emb_v[k, 4, r, sl] = plsc.load_gather(ctab_v, [gb])
            emb_v[k, 5, r, sl] = plsc.load_gather(ctab_v, [gb + 1])
            emb_v[k, 6, r, sl] = plsc.load_gather(ctab_v, [ob])
            for d in range(1, 8):
                emb_v[k, 6 + d, r, sl] = plsc.load_gather(ctab_v, [ob + d])

    def per_turn(g, carry):
        for k in range(NBUF):
            t = g * NBUF + k

            k2 = (k + 1) % NBUF

            @pl.when(jnp.logical_and(t >= 2, t - 2 < NITER))
            def _():
                rs, bs = _block_slices(base_r, t - 2)
                _drain_out(out_hbm, emb_v, in_v, k2, rs, bs, so[k2])

            @pl.when(t + 1 < NITER)
            def _():
                rs, bs = _block_slices(base_r, t + 1)
                _fire_in(x_hbm, in_v, k2, rs, bs, si[k2])

            @pl.when(t < NITER)
            def _():
                rs, bs = _block_slices(base_r, t)
                _wait_in(x_hbm, in_v, k, rs, bs, si[k])
                compute(k)
                _fire_out(out_hbm, emb_v, in_v, k, rs, bs, so[k])

        return carry

    lax.fori_loop(0, (NITER + NBUF) // NBUF, per_turn, 0)


@jax.jit
def kernel(x, age_table, gender_table, occupation_table):
    x_t = jnp.transpose(x.reshape(B, C_IN, NM), (1, 2, 0))
    ctab = jnp.concatenate([age_table.reshape(-1), gender_table.reshape(-1),
                            occupation_table.reshape(-1)])
    mesh = plsc.VectorSubcoreMesh(core_axis_name="c", subcore_axis_name="s",
                                  num_cores=NC, num_subcores=NS)
    out_t = pl.kernel(
        _sc_body,
        out_type=jax.ShapeDtypeStruct((C_OUT, NM, B), jnp.float32),
        mesh=mesh,
        scratch_types=[
            pltpu.VMEM((CTAB,), jnp.float32),
            pltpu.VMEM((NBUF, C_IN, RSUB, BSUB), jnp.float32),
            pltpu.VMEM((NBUF, 14, RSUB, BSUB), jnp.float32),
            pltpu.SemaphoreType.DMA,
            pltpu.SemaphoreType.DMA,
            pltpu.SemaphoreType.DMA,
            pltpu.SemaphoreType.DMA,
            pltpu.SemaphoreType.DMA,
            pltpu.SemaphoreType.DMA,
        ],
        compiler_params=pltpu.CompilerParams(use_tc_tiling_on_sc=True,
                                             needs_layout_passes=False),
    )(x_t, ctab)
    return jnp.transpose(out_t, (2, 0, 1)).reshape(B, C_OUT, 32, 32)
